# baseline (speedup 1.0000x reference)
import dataclasses
import functools

import jax
import jax.numpy as jnp
from jax import lax
from jax.experimental import pallas as pl
from jax.experimental.pallas import tpu as pltpu
from jax.experimental.pallas import tpu_sc as plsc

N = 10000
E = 160000
D = 256
DH = 128

NUM_SC = 2
NUM_TILES = 16
CHUNK = 128
BLK_CHUNKS = 8
NUM_BLKS = 10
CHUNKS_PER_TILE = NUM_BLKS * BLK_CHUNKS
E_PAD = NUM_TILES * CHUNKS_PER_TILE * CHUNK
N_PAD = 10112
ROWS_PER_TILE = N_PAD // NUM_TILES

BLK = 400
N_DEG = 10400


def _sc_aggregate(x_pair, srcs, dsts, zacc):
    mesh = plsc.VectorSubcoreMesh(core_axis_name="c", subcore_axis_name="s")
    cp = pltpu.CompilerParams()
    if "needs_layout_passes" in pltpu.CompilerParams.__dataclass_fields__:
        cp = dataclasses.replace(cp, needs_layout_passes=False)

    @functools.partial(
        pl.kernel,
        compiler_params=cp,
        out_type=[
            jax.ShapeDtypeStruct((NUM_SC, N_PAD, DH), jnp.float32),
            jax.ShapeDtypeStruct((NUM_TILES, N_PAD), jnp.float32),
        ],
        mesh=mesh,
        scratch_types=[
            pltpu.VMEM((2, BLK_CHUNKS, CHUNK), jnp.int32),
            pltpu.VMEM((2, BLK_CHUNKS, CHUNK), jnp.int32),
            pltpu.VMEM((2, CHUNK, DH), jnp.float32),
            pltpu.VMEM((N_PAD,), jnp.float32),
            pltpu.VMEM_SHARED((N_PAD, DH), jnp.float32),
            pltpu.SemaphoreType.DMA,
            pltpu.SemaphoreType.DMA,
            pltpu.SemaphoreType.DMA,
            pltpu.SemaphoreType.DMA,
            pltpu.SemaphoreType.DMA,
            pltpu.SemaphoreType.DMA,
        ],
    )
    def agg(xp, srcs_h, dsts_h, zacc_h, sum_out, deg_out,
            src_v, dst_v, rows_v, deg_v, acc_sh, gs0, gs1, ss0, ss1,
            is0, is1):
        c = lax.axis_index("c")
        s = lax.axis_index("s")
        row0 = s * ROWS_PER_TILE
        ones16 = jnp.ones((16,), jnp.float32)
        gsems = (gs0, gs1)
        ssems = (ss0, ss1)

        def start_gather(sl, i, b):
            pltpu.async_copy(xp.at[src_v.at[sl, i]], rows_v.at[b], gsems[b])

        def wait_gather(b):
            pltpu.make_async_copy(
                xp.at[src_v.at[0, 0]], rows_v.at[b], gsems[b]).wait()

        def start_scatter(sl, i, b):
            pltpu.async_copy(rows_v.at[b], acc_sh.at[dst_v.at[sl, i]],
                             ssems[b], add=True)

        def wait_scatter(b):
            pltpu.make_async_copy(
                rows_v.at[b], acc_sh.at[dst_v.at[0, 0]], ssems[b]).wait()

        def start_idx_load(blk, sl):
            pltpu.async_copy(srcs_h.at[c, s, blk], src_v.at[sl], is0)
            pltpu.async_copy(dsts_h.at[s, blk], dst_v.at[sl], is1)

        def wait_idx_load():
            pltpu.make_async_copy(
                srcs_h.at[c, s, 0], src_v.at[0], is0).wait()
            pltpu.make_async_copy(
                dsts_h.at[s, 0], dst_v.at[0], is1).wait()

        pltpu.sync_copy(zacc_h, acc_sh.at[pl.ds(row0, ROWS_PER_TILE)])

        @pl.when(c == 0)
        def _():
            @pl.loop(0, N_PAD // 16)
            def _(i):
                deg_v[pl.ds(i * 16, 16)] = jnp.zeros((16,), jnp.float32)

        plsc.subcore_barrier()

        pltpu.sync_copy(srcs_h.at[c, s, 0], src_v.at[0])
        pltpu.sync_copy(dsts_h.at[s, 0], dst_v.at[0])
        start_gather(0, 0, 0)

        @pl.loop(0, NUM_BLKS, step=2)
        def _(blk0):
            for u in (0, 1):
                for i in range(BLK_CHUNKS):
                    b = i % 2
                    wait_gather(b)
                    if i == 2:
                        if u == 0:
                            start_idx_load(blk0 + 1, 1)
                        else:
                            @pl.when(blk0 < NUM_BLKS - 2)
                            def _():
                                start_idx_load(blk0 + 2, 0)
                    if i < BLK_CHUNKS - 1:
                        if u == 0 and i == 0:
                            @pl.when(blk0 > 0)
                            def _():
                                wait_scatter(1)
                        else:
                            wait_scatter(1 - b)
                        start_gather(u, i + 1, 1 - b)
                    else:
                        if u == 0:
                            wait_idx_load()
                            wait_scatter(1 - b)
                            start_gather(1, 0, 1 - b)
                        else:
                            @pl.when(blk0 < NUM_BLKS - 2)
                            def _():
                                wait_idx_load()
                                wait_scatter(1 - b)
                                start_gather(0, 0, 1 - b)
                    start_scatter(u, i, b)

                    @pl.when(c == 0)
                    def _():
                        @pl.loop(0, CHUNK // 16)
                        def _(k):
                            idx = dst_v[u, i, pl.ds(k * 16, 16)]
                            plsc.addupdate_scatter(deg_v, [idx], ones16)

        wait_scatter(0)
        wait_scatter(1)

        @pl.when(c == 0)
        def _():
            pltpu.sync_copy(deg_v, deg_out.at[s])

        plsc.subcore_barrier()

        pltpu.sync_copy(acc_sh.at[pl.ds(row0, ROWS_PER_TILE)],
                        sum_out.at[c, pl.ds(row0, ROWS_PER_TILE)])

    return agg(x_pair, srcs, dsts, zacc)


def _tc_root_body(x_ref, wrT_ref, b_ref, o_ref):
    dn = (((1,), (0,)), ((), ()))
    o_ref[...] = lax.dot_general(
        x_ref[...], wrT_ref[...], dn,
        preferred_element_type=jnp.float32) + b_ref[...]


def _tc_root(x, wrT, b2):
    return pl.pallas_call(
        _tc_root_body,
        grid=(N // BLK,),
        in_specs=[
            pl.BlockSpec((BLK, D), lambda i: (i, 0)),
            pl.BlockSpec((D, D), lambda i: (0, 0)),
            pl.BlockSpec((1, D), lambda i: (0, 0)),
        ],
        out_specs=pl.BlockSpec((BLK, D), lambda i: (i, 0)),
        out_shape=jax.ShapeDtypeStruct((N, D), jnp.float32),
    )(x, wrT, b2)


def _tc_body(s_ref, d_ref, yr_ref, wlT_ref, o_ref):
    deg = jnp.sum(d_ref[...], axis=1).reshape(BLK, 1)
    inv = 1.0 / jnp.maximum(deg, 1.0)
    mlo = s_ref[0] * inv
    mhi = s_ref[1] * inv
    wlT = wlT_ref[...]
    dn = (((1,), (0,)), ((), ()))
    out = lax.dot_general(mlo, wlT[:DH, :], dn,
                          preferred_element_type=jnp.float32)
    out += lax.dot_general(mhi, wlT[DH:, :], dn,
                           preferred_element_type=jnp.float32)
    o_ref[...] = out + yr_ref[...]


def _tc_finish(summed, deg16, y_r, wlT):
    return pl.pallas_call(
        _tc_body,
        grid=(N // BLK,),
        in_specs=[
            pl.BlockSpec((NUM_SC, BLK, DH), lambda i: (0, i, 0)),
            pl.BlockSpec((BLK, NUM_TILES), lambda i: (i, 0)),
            pl.BlockSpec((BLK, D), lambda i: (i, 0)),
            pl.BlockSpec((D, D), lambda i: (0, 0)),
        ],
        out_specs=pl.BlockSpec((BLK, D), lambda i: (i, 0)),
        out_shape=jax.ShapeDtypeStruct((N, D), jnp.float32),
    )(summed, deg16, y_r, wlT)


def kernel(x, edge_index, W_l, b_l, W_r):
    src = edge_index[0].astype(jnp.int32)
    dst = edge_index[1].astype(jnp.int32)

    x_pair = x.reshape(2 * N, DH)

    pad = E_PAD - E
    pad_src = (jnp.arange(pad, dtype=jnp.int32) * 97) % N
    pad_dst = N + (jnp.arange(pad, dtype=jnp.int32) % (N_PAD - N))
    src_p = jnp.concatenate([src, pad_src]) * 2
    dst_p = jnp.concatenate([dst, pad_dst])
    srcs = jnp.stack([src_p, src_p + 1]).reshape(
        NUM_SC, NUM_TILES, NUM_BLKS, BLK_CHUNKS, CHUNK)
    dsts = dst_p.reshape(NUM_TILES, NUM_BLKS, BLK_CHUNKS, CHUNK)
    zacc = jnp.zeros((ROWS_PER_TILE, DH), jnp.float32)

    y_r = _tc_root(x, W_r.T, b_l.reshape(1, D))
    summed, deg16 = _sc_aggregate(x_pair, srcs, dsts, zacc)
    deg16 = jnp.pad(deg16.T, ((0, N_DEG - N_PAD), (0, 0)))

    return _tc_finish(summed, deg16, y_r, W_l.T)


# speedup vs baseline: 1.0623x; 1.0623x over previous
import dataclasses
import functools

import jax
import jax.numpy as jnp
from jax import lax
from jax.experimental import pallas as pl
from jax.experimental.pallas import tpu as pltpu
from jax.experimental.pallas import tpu_sc as plsc

N = 10000
E = 160000
D = 256
DH = 128

NUM_SC = 2
NUM_TILES = 16
CHUNK = 128
BLK_CHUNKS = 8
NUM_BLKS = 10
CHUNKS_PER_TILE = NUM_BLKS * BLK_CHUNKS
E_PAD = NUM_TILES * CHUNKS_PER_TILE * CHUNK
N_PAD = 10112
ROWS_PER_TILE = N_PAD // NUM_TILES

BLK = 2000


def _sc_aggregate(x_pair, srcs, dsts, zacc):
    mesh = plsc.VectorSubcoreMesh(core_axis_name="c", subcore_axis_name="s")
    cp = pltpu.CompilerParams()
    if "needs_layout_passes" in pltpu.CompilerParams.__dataclass_fields__:
        cp = dataclasses.replace(cp, needs_layout_passes=False)

    @functools.partial(
        pl.kernel,
        compiler_params=cp,
        out_type=[
            jax.ShapeDtypeStruct((NUM_SC, N_PAD, DH), jnp.float32),
            jax.ShapeDtypeStruct((NUM_TILES, N_PAD), jnp.float32),
        ],
        mesh=mesh,
        scratch_types=[
            pltpu.VMEM((2, BLK_CHUNKS * CHUNK), jnp.int32),
            pltpu.VMEM((2, BLK_CHUNKS, CHUNK), jnp.int32),
            pltpu.VMEM((2, CHUNK, DH), jnp.float32),
            pltpu.VMEM((N_PAD,), jnp.float32),
            pltpu.VMEM_SHARED((N_PAD, DH), jnp.float32),
            pltpu.SemaphoreType.DMA,
            pltpu.SemaphoreType.DMA,
            pltpu.SemaphoreType.DMA,
            pltpu.SemaphoreType.DMA,
            pltpu.SemaphoreType.DMA,
            pltpu.SemaphoreType.DMA,
        ],
    )
    def agg(xp, srcs_h, dsts_h, zacc_h, sum_out, deg_out,
            src_v, dst_v, rows_v, deg_v, acc_sh, gs0, gs1, ss0, ss1,
            is0, is1):
        c = lax.axis_index("c")
        s = lax.axis_index("s")
        row0 = s * ROWS_PER_TILE
        ones16 = jnp.ones((16,), jnp.float32)
        gsems = (gs0, gs1)
        ssems = (ss0, ss1)

        def start_gather(sl, i, b):
            pltpu.async_copy(xp.at[src_v.at[sl, pl.ds(i * CHUNK, CHUNK)]],
                             rows_v.at[b], gsems[b])

        def wait_gather(b):
            pltpu.make_async_copy(
                xp.at[src_v.at[0, pl.ds(0, CHUNK)]], rows_v.at[b],
                gsems[b]).wait()

        def fixup_src(sl):
            @pl.when(c == 1)
            def _():
                @pl.loop(0, BLK_CHUNKS * CHUNK // 16)
                def _(k):
                    sl16 = (sl, pl.ds(k * 16, 16))
                    src_v[sl16] = src_v[sl16] + 1

        def start_scatter(sl, i, b):
            pltpu.async_copy(rows_v.at[b], acc_sh.at[dst_v.at[sl, i]],
                             ssems[b], add=True)

        def wait_scatter(b):
            pltpu.make_async_copy(
                rows_v.at[b], acc_sh.at[dst_v.at[0, 0]], ssems[b]).wait()

        def start_idx_load(blk, sl):
            pltpu.async_copy(srcs_h.at[s, blk], src_v.at[sl], is0)
            pltpu.async_copy(dsts_h.at[s, blk], dst_v.at[sl], is1)

        def wait_idx_load():
            pltpu.make_async_copy(
                srcs_h.at[0, 0], src_v.at[0], is0).wait()
            pltpu.make_async_copy(
                dsts_h.at[0, 0], dst_v.at[0], is1).wait()

        pltpu.sync_copy(zacc_h, acc_sh.at[pl.ds(row0, ROWS_PER_TILE)])

        @pl.when(c == 0)
        def _():
            @pl.loop(0, N_PAD // 16)
            def _(i):
                deg_v[pl.ds(i * 16, 16)] = jnp.zeros((16,), jnp.float32)

        plsc.subcore_barrier()

        pltpu.sync_copy(srcs_h.at[s, 0], src_v.at[0])
        pltpu.sync_copy(dsts_h.at[s, 0], dst_v.at[0])
        fixup_src(0)
        start_gather(0, 0, 0)

        @pl.loop(0, NUM_BLKS, step=2)
        def _(blk0):
            for u in (0, 1):
                for i in range(BLK_CHUNKS):
                    b = i % 2
                    wait_gather(b)
                    if i == 2:
                        if u == 0:
                            start_idx_load(blk0 + 1, 1)
                        else:
                            @pl.when(blk0 < NUM_BLKS - 2)
                            def _():
                                start_idx_load(blk0 + 2, 0)
                    if i < BLK_CHUNKS - 1:
                        if u == 0 and i == 0:
                            @pl.when(blk0 > 0)
                            def _():
                                wait_scatter(1)
                        else:
                            wait_scatter(1 - b)
                        start_gather(u, i + 1, 1 - b)
                    else:
                        if u == 0:
                            wait_idx_load()
                            fixup_src(1)
                            wait_scatter(1 - b)
                            start_gather(1, 0, 1 - b)
                        else:
                            @pl.when(blk0 < NUM_BLKS - 2)
                            def _():
                                wait_idx_load()
                                fixup_src(0)
                                wait_scatter(1 - b)
                                start_gather(0, 0, 1 - b)
                    start_scatter(u, i, b)

                    @pl.when(c == 0)
                    def _():
                        @pl.loop(0, CHUNK // 16)
                        def _(k):
                            idx = dst_v[u, i, pl.ds(k * 16, 16)]
                            plsc.addupdate_scatter(deg_v, [idx], ones16)

        wait_scatter(0)
        wait_scatter(1)

        @pl.when(c == 0)
        def _():
            pltpu.sync_copy(deg_v, deg_out.at[s])

        plsc.subcore_barrier()

        pltpu.sync_copy(acc_sh.at[pl.ds(row0, ROWS_PER_TILE)],
                        sum_out.at[c, pl.ds(row0, ROWS_PER_TILE)])

    return agg(x_pair, srcs, dsts, zacc)


def _tc_root_body(x_ref, wrT_ref, b_ref, o_ref):
    dn = (((1,), (0,)), ((), ()))
    o_ref[...] = lax.dot_general(
        x_ref[...], wrT_ref[...], dn,
        preferred_element_type=jnp.float32) + b_ref[...]


def _tc_root(x, wrT, b2):
    return pl.pallas_call(
        _tc_root_body,
        grid=(N // BLK,),
        in_specs=[
            pl.BlockSpec((BLK, D), lambda i: (i, 0)),
            pl.BlockSpec((D, D), lambda i: (0, 0)),
            pl.BlockSpec((1, D), lambda i: (0, 0)),
        ],
        out_specs=pl.BlockSpec((BLK, D), lambda i: (i, 0)),
        out_shape=jax.ShapeDtypeStruct((N, D), jnp.float32),
    )(x, wrT, b2)


def _tc_body(s_ref, d_ref, yr_ref, wlT_ref, o_ref):
    deg = jnp.sum(d_ref[...], axis=1).reshape(BLK, 1)
    inv = 1.0 / jnp.maximum(deg, 1.0)
    mlo = s_ref[0] * inv
    mhi = s_ref[1] * inv
    wlT = wlT_ref[...]
    dn = (((1,), (0,)), ((), ()))
    out = lax.dot_general(mlo, wlT[:DH, :], dn,
                          preferred_element_type=jnp.float32)
    out += lax.dot_general(mhi, wlT[DH:, :], dn,
                           preferred_element_type=jnp.float32)
    o_ref[...] = out + yr_ref[...]


def _tc_finish(summed, deg16, y_r, wlT):
    return pl.pallas_call(
        _tc_body,
        grid=(N // BLK,),
        in_specs=[
            pl.BlockSpec((NUM_SC, BLK, DH), lambda i: (0, i, 0)),
            pl.BlockSpec((BLK, NUM_TILES), lambda i: (i, 0)),
            pl.BlockSpec((BLK, D), lambda i: (i, 0)),
            pl.BlockSpec((D, D), lambda i: (0, 0)),
        ],
        out_specs=pl.BlockSpec((BLK, D), lambda i: (i, 0)),
        out_shape=jax.ShapeDtypeStruct((N, D), jnp.float32),
    )(summed, deg16, y_r, wlT)


def kernel(x, edge_index, W_l, b_l, W_r):
    src = edge_index[0].astype(jnp.int32)
    dst = edge_index[1].astype(jnp.int32)

    x_pair = x.reshape(2 * N, DH)

    pad = E_PAD - E
    pad_src = (jnp.arange(pad, dtype=jnp.int32) * 97) % N
    pad_dst = N + (jnp.arange(pad, dtype=jnp.int32) % (N_PAD - N))
    src_p = jnp.concatenate([src, pad_src]) * 2
    dst_p = jnp.concatenate([dst, pad_dst])
    srcs = src_p.reshape(NUM_TILES, NUM_BLKS, BLK_CHUNKS * CHUNK)
    dsts = dst_p.reshape(NUM_TILES, NUM_BLKS, BLK_CHUNKS, CHUNK)
    zacc = jnp.zeros((ROWS_PER_TILE, DH), jnp.float32)

    y_r = _tc_root(x, W_r.T, b_l.reshape(1, D))
    summed, deg16 = _sc_aggregate(x_pair, srcs, dsts, zacc)
    deg16 = deg16.T

    return _tc_finish(summed, deg16, y_r, W_l.T)


# speedup vs baseline: 1.0809x; 1.0175x over previous
import dataclasses
import functools

import jax
import jax.numpy as jnp
from jax import lax
from jax.experimental import pallas as pl
from jax.experimental.pallas import tpu as pltpu
from jax.experimental.pallas import tpu_sc as plsc

N = 10000
E = 160000
D = 256
DH = 128

NUM_SC = 2
NUM_TILES = 16
CHUNK = 128
BLK_CHUNKS = 8
NUM_BLKS = 10
CHUNKS_PER_TILE = NUM_BLKS * BLK_CHUNKS
E_PAD = NUM_TILES * CHUNKS_PER_TILE * CHUNK
N_PAD = 10112
ROWS_PER_TILE = N_PAD // NUM_TILES

BLK = 2000


def _sc_aggregate(x_pair, srcs, dsts, zacc):
    mesh = plsc.VectorSubcoreMesh(core_axis_name="c", subcore_axis_name="s")
    cp = pltpu.CompilerParams()
    if "needs_layout_passes" in pltpu.CompilerParams.__dataclass_fields__:
        cp = dataclasses.replace(cp, needs_layout_passes=False)

    @functools.partial(
        pl.kernel,
        compiler_params=cp,
        out_type=[
            jax.ShapeDtypeStruct((NUM_SC, N_PAD, DH), jnp.float32),
            jax.ShapeDtypeStruct((NUM_TILES, N_PAD), jnp.float32),
        ],
        mesh=mesh,
        scratch_types=[
            pltpu.VMEM((2, BLK_CHUNKS * CHUNK), jnp.int32),
            pltpu.VMEM((2, BLK_CHUNKS * CHUNK), jnp.int32),
            pltpu.VMEM((2, CHUNK, DH), jnp.float32),
            pltpu.VMEM((N_PAD,), jnp.float32),
            pltpu.VMEM_SHARED((N_PAD, DH), jnp.float32),
            pltpu.SemaphoreType.DMA,
            pltpu.SemaphoreType.DMA,
            pltpu.SemaphoreType.DMA,
            pltpu.SemaphoreType.DMA,
            pltpu.SemaphoreType.DMA,
            pltpu.SemaphoreType.DMA,
        ],
    )
    def agg(xp, srcs_h, dsts_h, zacc_h, sum_out, deg_out,
            src_v, dst_v, rows_v, deg_v, acc_sh, gs0, gs1, ss0, ss1,
            is0, is1):
        c = lax.axis_index("c")
        s = lax.axis_index("s")
        row0 = s * ROWS_PER_TILE
        ones16 = jnp.ones((16,), jnp.float32)
        gsems = (gs0, gs1)
        ssems = (ss0, ss1)

        def start_gather(sl, i, b):
            pltpu.async_copy(xp.at[src_v.at[sl, pl.ds(i * CHUNK, CHUNK)]],
                             rows_v.at[b], gsems[b])

        def wait_gather(b):
            pltpu.make_async_copy(
                xp.at[src_v.at[0, pl.ds(0, CHUNK)]], rows_v.at[b],
                gsems[b]).wait()

        def fixup_src(sl):
            @pl.when(c == 1)
            def _():
                @pl.loop(0, BLK_CHUNKS * CHUNK // 16)
                def _(k):
                    sl16 = (sl, pl.ds(k * 16, 16))
                    src_v[sl16] = src_v[sl16] + 1

        def start_scatter(sl, i, b):
            pltpu.async_copy(
                rows_v.at[b],
                acc_sh.at[dst_v.at[sl, pl.ds(i * CHUNK, CHUNK)]],
                ssems[b], add=True)

        def wait_scatter(b):
            pltpu.make_async_copy(
                rows_v.at[b], acc_sh.at[dst_v.at[0, pl.ds(0, CHUNK)]],
                ssems[b]).wait()

        blk_len = BLK_CHUNKS * CHUNK

        def start_idx_load(blk, sl):
            off = (s * NUM_BLKS + blk) * blk_len
            pltpu.async_copy(srcs_h.at[pl.ds(off, blk_len)],
                             src_v.at[sl], is0)
            pltpu.async_copy(dsts_h.at[pl.ds(off, blk_len)],
                             dst_v.at[sl], is1)

        def wait_idx_load():
            pltpu.make_async_copy(
                srcs_h.at[pl.ds(0, blk_len)], src_v.at[0], is0).wait()
            pltpu.make_async_copy(
                dsts_h.at[pl.ds(0, blk_len)], dst_v.at[0], is1).wait()

        pltpu.sync_copy(zacc_h, acc_sh.at[pl.ds(row0, ROWS_PER_TILE)])

        @pl.when(c == 0)
        def _():
            @pl.loop(0, N_PAD // 16)
            def _(i):
                deg_v[pl.ds(i * 16, 16)] = jnp.zeros((16,), jnp.float32)

        plsc.subcore_barrier()

        off0 = s * NUM_BLKS * blk_len
        pltpu.sync_copy(srcs_h.at[pl.ds(off0, blk_len)], src_v.at[0])
        pltpu.sync_copy(dsts_h.at[pl.ds(off0, blk_len)], dst_v.at[0])
        fixup_src(0)
        start_gather(0, 0, 0)

        @pl.loop(0, NUM_BLKS, step=2)
        def _(blk0):
            for u in (0, 1):
                for i in range(BLK_CHUNKS):
                    b = i % 2
                    wait_gather(b)
                    if i == 2:
                        if u == 0:
                            start_idx_load(blk0 + 1, 1)
                        else:
                            @pl.when(blk0 < NUM_BLKS - 2)
                            def _():
                                start_idx_load(blk0 + 2, 0)
                    if i < BLK_CHUNKS - 1:
                        if u == 0 and i == 0:
                            @pl.when(blk0 > 0)
                            def _():
                                wait_scatter(1)
                        else:
                            wait_scatter(1 - b)
                        start_gather(u, i + 1, 1 - b)
                    else:
                        if u == 0:
                            wait_idx_load()
                            fixup_src(1)
                            wait_scatter(1 - b)
                            start_gather(1, 0, 1 - b)
                        else:
                            @pl.when(blk0 < NUM_BLKS - 2)
                            def _():
                                wait_idx_load()
                                fixup_src(0)
                                wait_scatter(1 - b)
                                start_gather(0, 0, 1 - b)
                    start_scatter(u, i, b)

                    @pl.when(c == 0)
                    def _():
                        @pl.loop(0, CHUNK // 16)
                        def _(k):
                            idx = dst_v[u, pl.ds(i * CHUNK + k * 16, 16)]
                            plsc.addupdate_scatter(deg_v, [idx], ones16)

        wait_scatter(0)
        wait_scatter(1)

        @pl.when(c == 0)
        def _():
            pltpu.sync_copy(deg_v, deg_out.at[s])

        plsc.subcore_barrier()

        pltpu.sync_copy(acc_sh.at[pl.ds(row0, ROWS_PER_TILE)],
                        sum_out.at[c, pl.ds(row0, ROWS_PER_TILE)])

    return agg(x_pair, srcs, dsts, zacc)


def _tc_root_body(x_ref, wrT_ref, b_ref, o_ref):
    dn = (((1,), (0,)), ((), ()))
    o_ref[...] = lax.dot_general(
        x_ref[...], wrT_ref[...], dn,
        preferred_element_type=jnp.float32) + b_ref[...]


def _tc_root(x, wrT, b2):
    return pl.pallas_call(
        _tc_root_body,
        grid=(N // BLK,),
        in_specs=[
            pl.BlockSpec((BLK, D), lambda i: (i, 0)),
            pl.BlockSpec((D, D), lambda i: (0, 0)),
            pl.BlockSpec((1, D), lambda i: (0, 0)),
        ],
        out_specs=pl.BlockSpec((BLK, D), lambda i: (i, 0)),
        out_shape=jax.ShapeDtypeStruct((N, D), jnp.float32),
    )(x, wrT, b2)


def _tc_body(s_ref, d_ref, yr_ref, wlT_ref, o_ref):
    deg = jnp.sum(d_ref[...], axis=1).reshape(BLK, 1)
    inv = 1.0 / jnp.maximum(deg, 1.0)
    mlo = s_ref[0] * inv
    mhi = s_ref[1] * inv
    wlT = wlT_ref[...]
    dn = (((1,), (0,)), ((), ()))
    out = lax.dot_general(mlo, wlT[:DH, :], dn,
                          preferred_element_type=jnp.float32)
    out += lax.dot_general(mhi, wlT[DH:, :], dn,
                           preferred_element_type=jnp.float32)
    o_ref[...] = out + yr_ref[...]


def _tc_finish(summed, deg16, y_r, wlT):
    return pl.pallas_call(
        _tc_body,
        grid=(N // BLK,),
        in_specs=[
            pl.BlockSpec((NUM_SC, BLK, DH), lambda i: (0, i, 0)),
            pl.BlockSpec((BLK, NUM_TILES), lambda i: (i, 0)),
            pl.BlockSpec((BLK, D), lambda i: (i, 0)),
            pl.BlockSpec((D, D), lambda i: (0, 0)),
        ],
        out_specs=pl.BlockSpec((BLK, D), lambda i: (i, 0)),
        out_shape=jax.ShapeDtypeStruct((N, D), jnp.float32),
    )(summed, deg16, y_r, wlT)


def kernel(x, edge_index, W_l, b_l, W_r):
    src = edge_index[0].astype(jnp.int32)
    dst = edge_index[1].astype(jnp.int32)

    x_pair = x.reshape(2 * N, DH)

    pad = E_PAD - E
    pad_src = (jnp.arange(pad, dtype=jnp.int32) * 97) % N
    pad_dst = N + (jnp.arange(pad, dtype=jnp.int32) % (N_PAD - N))
    srcs = jnp.concatenate([src, pad_src]) * 2
    dsts = jnp.concatenate([dst, pad_dst])
    zacc = jnp.zeros((ROWS_PER_TILE, DH), jnp.float32)

    y_r = _tc_root(x, W_r.T, b_l.reshape(1, D))
    summed, deg16 = _sc_aggregate(x_pair, srcs, dsts, zacc)
    deg16 = deg16.T

    return _tc_finish(summed, deg16, y_r, W_l.T)
